# Initial kernel scaffold; baseline (speedup 1.0000x reference)
import jax
import jax.numpy as jnp
from jax.experimental import pallas as pl


def kernel(node_embeds, basis, comp, root, bias, ep1_w1, ep1_b1, ep1_w2, ep1_b2, ep2_w, ep2_b, pp1_w1, pp1_b1, pp1_w2, pp1_b2, pp2_w, pp2_b, edge_index, edge_type, entity_ids, rec_labels):
    raise NotImplementedError("write your pallas kernel here")


# speedup vs baseline: 70.4778x; 70.4778x over previous
import functools

import jax
import jax.numpy as jnp
from jax import lax
from jax.experimental import pallas as pl
from jax.experimental.pallas import tpu as pltpu, tpu_sc as plsc

HID = 768
EH = 384
NL = 12
NB = 2
NH = 12
HD = 64
NE = 10000
E = 320000
R = 9
NBASES = 8
B = 16
LENT = 32
S = B * LENT

NC = 2
NS = 16
NW = NC * NS
EPT = E // NW
K = 64
ROWS = R * S
OPT = ROWS // NS
RPH = OPT // 2
CAP = EPT + 16


def _sc_body(espk_hbm, edst_hbm, ids_hbm, x_hbm,
             a2_hbm, cnt_hbm, xg_hbm, rep_hbm, lists_hbm, lcnt_hbm,
             slot_v, ids_v, esrc_v, edst_v,
             idxg2, rows_v, cnt_v, acc_v, pend_src, pend_row,
             idx16_v, rep_v, c128_v, sem):
    cid = lax.axis_index("c")
    sid = lax.axis_index("s")
    wid = sid * NC + cid
    lid = cid * NS + sid
    i32 = jnp.int32
    f32 = jnp.float32
    zero16 = jnp.zeros((16,), f32)
    iota16 = lax.broadcasted_iota(i32, (16,), 0)

    pltpu.sync_copy(ids_hbm, ids_v)

    def _init_slot(i, _):
        slot_v[pl.ds(i * 16, 16)] = jnp.full((16,), -1, i32)
        return _
    lax.fori_loop(0, NE // 16, _init_slot, None)

    def _scatter_ids(i, _):
        idv = ids_v[pl.ds(i * 16, 16)]
        plsc.store_scatter(slot_v, [idv], iota16 + i * 16)
        return _
    lax.fori_loop(0, S // 16, _scatter_ids, None)

    def _zero_cnt(i, _):
        cnt_v[pl.ds(i * 16, 16)] = zero16
        return _
    lax.fori_loop(0, ROWS // 16, _zero_cnt, None)

    myids = ids_v[pl.ds(wid * 16, 16)]
    idx16_v[...] = myids
    pltpu.async_copy(x_hbm.at[idx16_v], rows_v.at[pl.ds(0, 16)], sem).wait()
    pltpu.sync_copy(rows_v.at[pl.ds(0, 16)], xg_hbm.at[pl.ds(wid * 16, 16)])
    rep_v[...] = plsc.load_gather(slot_v, [myids])
    pltpu.sync_copy(rep_v, rep_hbm.at[pl.ds(wid * 16, 16)])

    pltpu.sync_copy(espk_hbm.at[pl.ds(wid * EPT, EPT)], esrc_v.at[pl.ds(0, EPT)])
    pltpu.sync_copy(edst_hbm.at[pl.ds(wid * EPT, EPT)], edst_v.at[pl.ds(0, EPT)])

    ones16 = jnp.full((16,), 1.0, f32)

    def _scan(i, ca):
        st16 = esrc_v[pl.ds(i * 16, 16)]
        s16 = lax.shift_right_logical(st16, 4)
        t16 = jnp.bitwise_and(st16, 15)
        d16 = edst_v[pl.ds(i * 16, 16)]
        sl16 = plsc.load_gather(slot_v, [d16])
        m = sl16 >= 0
        slotidx = t16 * S + sl16
        plsc.addupdate_scatter(cnt_v, [jnp.where(m, slotidx, 0)], ones16,
                               mask=m)
        plsc.store_compressed(esrc_v.at[pl.ds(ca, 16)], s16, mask=m)
        plsc.store_compressed(edst_v.at[pl.ds(ca, 16)], slotidx, mask=m)
        return ca + jnp.max(plsc.all_reduce_population_count(m))
    ca = lax.fori_loop(0, EPT // 16, _scan, jnp.int32(0))

    pltpu.sync_copy(esrc_v, lists_hbm.at[lid, 0])
    pltpu.sync_copy(edst_v, lists_hbm.at[lid, 1])
    idx16_v[...] = jnp.full((16,), 1, i32) * ca
    pltpu.sync_copy(idx16_v.at[pl.ds(0, 8)], lcnt_hbm.at[pl.ds(lid * 8, 8)])
    pltpu.sync_copy(cnt_v, cnt_hbm.at[wid])
    plsc.subcore_barrier()

    pltpu.sync_copy(lcnt_hbm.at[pl.ds(cid * NS * 8, 128)], c128_v)

    def _flush(n, base):
        for j in range(K // 16):
            v = pend_src[pl.ds(j * 16, 16)]
            ok = (iota16 + j * 16) < n
            idxg2[0, pl.ds(j * 16, 16)] = jnp.where(ok, v, 0)
        pltpu.async_copy(x_hbm.at[idxg2.at[0]], rows_v, sem).wait()

        def _addone(e, _):
            @pl.when(e < n)
            def _():
                rl = jnp.max(plsc.load_gather(pend_row,
                                              [jnp.full((16,), e, i32)]))
                rl = rl - base
                for cg in range(EH // 16):
                    plsc.addupdate(acc_v.at[rl, pl.ds(cg * 16, 16)],
                                   rows_v[e, pl.ds(cg * 16, 16)])
            return _
        lax.fori_loop(0, K, _addone, None)

    def _pass(p, _):
        base = sid * OPT + p * RPH

        def _zacc(i, _):
            for cg in range(EH // 16):
                acc_v[i, pl.ds(cg * 16, 16)] = zero16
            return _
        lax.fori_loop(0, RPH, _zacc, None)

        def _tbody(t, pc):
            pltpu.sync_copy(lists_hbm.at[cid * NS + t, 0], esrc_v)
            pltpu.sync_copy(lists_hbm.at[cid * NS + t, 1], edst_v)
            nt = jnp.max(plsc.load_gather(c128_v,
                                          [jnp.full((16,), 8, i32) * t]))

            def _collect(i, pc):
                s16 = esrc_v[pl.ds(i * 16, 16)]
                g16 = edst_v[pl.ds(i * 16, 16)]
                mine = jnp.logical_and((i * 16 + iota16) < nt,
                                       jnp.logical_and(g16 >= base,
                                                       g16 < base + RPH))
                plsc.store_compressed(pend_src.at[pl.ds(pc, 16)], s16,
                                      mask=mine)
                plsc.store_compressed(pend_row.at[pl.ds(pc, 16)], g16,
                                      mask=mine)
                pc2 = pc + jnp.max(plsc.all_reduce_population_count(mine))

                @pl.when(pc2 >= K)
                def _():
                    _flush(jnp.int32(K), base)
                    pend_src[pl.ds(0, 16)] = pend_src[pl.ds(K, 16)]
                    pend_row[pl.ds(0, 16)] = pend_row[pl.ds(K, 16)]
                return jnp.where(pc2 >= K, pc2 - K, pc2)
            return lax.fori_loop(0, (nt + 15) // 16, _collect, pc)

        pc = lax.fori_loop(0, NS, _tbody, jnp.int32(0))
        _flush(pc, base)
        pltpu.sync_copy(acc_v, a2_hbm.at[pl.ds(cid * ROWS + base, RPH)])
        return _
    lax.fori_loop(0, 2, _pass, None)


@jax.jit
def _sc_gather(espk, edst, ids, x):
    mesh = plsc.VectorSubcoreMesh(core_axis_name="c", subcore_axis_name="s")
    f = pl.kernel(
        _sc_body,
        out_type=(
            jax.ShapeDtypeStruct((NC * ROWS, EH), jnp.float32),
            jax.ShapeDtypeStruct((NW, ROWS), jnp.float32),
            jax.ShapeDtypeStruct((S, EH), jnp.float32),
            jax.ShapeDtypeStruct((S,), jnp.int32),
            jax.ShapeDtypeStruct((NW, 2, CAP), jnp.int32),
            jax.ShapeDtypeStruct((NW * 8,), jnp.int32),
        ),
        mesh=mesh,
        scratch_types=(
            pltpu.VMEM((NE,), jnp.int32),
            pltpu.VMEM((S,), jnp.int32),
            pltpu.VMEM((CAP,), jnp.int32),
            pltpu.VMEM((CAP,), jnp.int32),
            pltpu.VMEM((1, K), jnp.int32),
            pltpu.VMEM((K, EH), jnp.float32),
            pltpu.VMEM((ROWS,), jnp.float32),
            pltpu.VMEM((RPH, EH), jnp.float32),
            pltpu.VMEM((K + 16,), jnp.int32),
            pltpu.VMEM((K + 16,), jnp.int32),
            pltpu.VMEM((16,), jnp.int32),
            pltpu.VMEM((16,), jnp.int32),
            pltpu.VMEM((128,), jnp.int32),
            pltpu.SemaphoreType.DMA,
        ),
        compiler_params=pltpu.CompilerParams(needs_layout_passes=False),
    )
    return f(espk, edst, ids, x)


def _t1_body(a2_ref, cnt_ref, xg_ref, rep_ref, basis_ref, comp_ref,
             root_ref, bias_ref, e11_ref, e1b1_ref, e12_ref, e1b2_ref,
             e2w_ref, e2b_ref, p11_ref, p1b1_ref, p12_ref, p1b2_ref,
             out_ref):
    f32 = jnp.float32
    a2 = a2_ref[0:ROWS, :] + a2_ref[ROWS:2 * ROWS, :]
    xg = xg_ref[...]
    msg = jnp.zeros((S, EH), f32)
    for r in range(R):
        cnt_r = jnp.sum(cnt_ref[:, r, :], axis=0)
        inv = 1.0 / jnp.maximum(cnt_r, 1.0)
        a_r = a2[r * S:(r + 1) * S, :] * inv[:, None]
        w_r = comp_ref[r, 0] * basis_ref[0]
        for b in range(1, NBASES):
            w_r = w_r + comp_ref[r, b] * basis_ref[b]
        msg = msg + jnp.dot(a_r, w_r, preferred_element_type=f32)
    ent = (jnp.dot(xg, root_ref[...], preferred_element_type=f32)
           + bias_ref[...] + msg + xg)
    h = jnp.maximum(jnp.dot(ent, e11_ref[...], preferred_element_type=f32)
                    + e1b1_ref[...], 0.0)
    ent = ent + jnp.dot(h, e12_ref[...], preferred_element_type=f32) + e1b2_ref[...]
    ent3 = jnp.dot(ent, e2w_ref[...], preferred_element_type=f32) + e2b_ref[...]
    onehot = (rep_ref[...] == lax.broadcasted_iota(jnp.int32, (S, S), 1)
              ).astype(f32)
    pe0 = jnp.dot(onehot, ent3, preferred_element_type=f32)
    h = jnp.maximum(jnp.dot(pe0, p11_ref[...], preferred_element_type=f32)
                    + p1b1_ref[...], 0.0)
    out_ref[...] = (pe0 + jnp.dot(h, p12_ref[...], preferred_element_type=f32)
                    + p1b2_ref[...])


def _t2_body(pe1_ref, w_ref, b_ref, out_ref):
    y = jnp.dot(pe1_ref[...], w_ref[...], preferred_element_type=jnp.float32)
    y = y + b_ref[0]
    for h in range(NH):
        out_ref[0, 0, :, h, :, :] = y[:, h * HD:(h + 1) * HD].reshape(B, LENT, HD)


def kernel(node_embeds, basis, comp, root, bias,
           ep1_w1, ep1_b1, ep1_w2, ep1_b2, ep2_w, ep2_b,
           pp1_w1, pp1_b1, pp1_w2, pp1_b2, pp2_w, pp2_b,
           edge_index, edge_type, entity_ids, rec_labels):
    ids = entity_ids.reshape(-1)
    espk = edge_index[0] * 16 + edge_type
    a2, cnt, xg, rep, _, _ = _sc_gather(espk, edge_index[1], ids, node_embeds)

    vspec = pl.BlockSpec(memory_space=pltpu.VMEM)
    pe1 = pl.pallas_call(
        _t1_body,
        in_specs=[vspec, vspec, vspec, vspec, vspec,
                  pl.BlockSpec(memory_space=pltpu.SMEM),
                  vspec, vspec, vspec, vspec, vspec, vspec, vspec, vspec,
                  vspec, vspec, vspec, vspec],
        out_specs=vspec,
        out_shape=jax.ShapeDtypeStruct((S, HID), jnp.float32),
    )(a2, cnt.reshape(NW, R, S), xg, rep.reshape(S, 1), basis, comp,
      root, bias.reshape(1, EH),
      ep1_w1, ep1_b1.reshape(1, -1), ep1_w2, ep1_b2.reshape(1, -1),
      ep2_w, ep2_b.reshape(1, -1),
      pp1_w1, pp1_b1.reshape(1, -1), pp1_w2, pp1_b2.reshape(1, -1))

    out = pl.pallas_call(
        _t2_body,
        grid=(NL * NB,),
        in_specs=[
            pl.BlockSpec((S, HID), lambda i: (0, 0)),
            pl.BlockSpec((HID, HID), lambda i: (0, i)),
            pl.BlockSpec((1, 1, HID), lambda i: (i, 0, 0)),
        ],
        out_specs=pl.BlockSpec((1, 1, B, NH, LENT, HD),
                               lambda i: (i // 2, i % 2, 0, 0, 0, 0)),
        out_shape=jax.ShapeDtypeStruct((NL, NB, B, NH, LENT, HD), jnp.float32),
    )(pe1, pp2_w, pp2_b.reshape(NL * NB, 1, HID))
    return out


# speedup vs baseline: 78.2052x; 1.1096x over previous
import functools

import jax
import jax.numpy as jnp
from jax import lax
from jax.experimental import pallas as pl
from jax.experimental.pallas import tpu as pltpu, tpu_sc as plsc

HID = 768
EH = 384
NL = 12
NB = 2
NH = 12
HD = 64
NE = 10000
E = 320000
R = 9
NBASES = 8
B = 16
LENT = 32
S = B * LENT

NC = 2
NS = 16
NW = NC * NS
EPT = E // NW
K = 64
ROWS = R * S
OPT = ROWS // NS
RPH = OPT // 2
CH = 1024
CAP = EPT + CH - EPT % CH


def _sc_body(espk_hbm, edst_hbm, ids_hbm, x_hbm,
             a2_hbm, cnt_hbm, xg_hbm, rep_hbm, lists_hbm, lcnt_hbm,
             slot_v, ids_v, esrc_v, edst_v,
             idxg2, rows_v, cnt_v, acc_v, pend_src, pend_row,
             idx16_v, rep_v, c128_v, sem):
    cid = lax.axis_index("c")
    sid = lax.axis_index("s")
    wid = sid * NC + cid
    lid = cid * NS + sid
    i32 = jnp.int32
    f32 = jnp.float32
    zero16 = jnp.zeros((16,), f32)
    iota16 = lax.broadcasted_iota(i32, (16,), 0)

    pltpu.sync_copy(ids_hbm, ids_v)

    def _init_slot(i, _):
        slot_v[pl.ds(i * 16, 16)] = jnp.full((16,), -1, i32)
        return _
    lax.fori_loop(0, NE // 16, _init_slot, None)

    def _scatter_ids(i, _):
        idv = ids_v[pl.ds(i * 16, 16)]
        plsc.store_scatter(slot_v, [idv], iota16 + i * 16)
        return _
    lax.fori_loop(0, S // 16, _scatter_ids, None)

    def _zero_cnt(i, _):
        cnt_v[pl.ds(i * 16, 16)] = zero16
        return _
    lax.fori_loop(0, ROWS // 16, _zero_cnt, None)

    myids = ids_v[pl.ds(wid * 16, 16)]
    idx16_v[...] = myids
    pltpu.async_copy(x_hbm.at[idx16_v], rows_v.at[pl.ds(0, 16)], sem).wait()
    pltpu.sync_copy(rows_v.at[pl.ds(0, 16)], xg_hbm.at[pl.ds(wid * 16, 16)])
    rep_v[...] = plsc.load_gather(slot_v, [myids])
    pltpu.sync_copy(rep_v, rep_hbm.at[pl.ds(wid * 16, 16)])

    pltpu.sync_copy(espk_hbm.at[pl.ds(wid * EPT, EPT)], esrc_v.at[pl.ds(0, EPT)])
    pltpu.sync_copy(edst_hbm.at[pl.ds(wid * EPT, EPT)], edst_v.at[pl.ds(0, EPT)])

    ones16 = jnp.full((16,), 1.0, f32)

    def _scan(i, ca):
        st16 = esrc_v[pl.ds(i * 16, 16)]
        s16 = lax.shift_right_logical(st16, 4)
        t16 = jnp.bitwise_and(st16, 15)
        d16 = edst_v[pl.ds(i * 16, 16)]
        sl16 = plsc.load_gather(slot_v, [d16])
        m = sl16 >= 0
        slotidx = t16 * S + sl16
        plsc.addupdate_scatter(cnt_v, [jnp.where(m, slotidx, 0)], ones16,
                               mask=m)
        plsc.store_compressed(esrc_v.at[pl.ds(ca, 16)], s16, mask=m)
        plsc.store_compressed(edst_v.at[pl.ds(ca, 16)], slotidx, mask=m)
        return ca + jnp.max(plsc.all_reduce_population_count(m))
    ca = lax.fori_loop(0, EPT // 16, _scan, jnp.int32(0))

    pltpu.sync_copy(esrc_v, lists_hbm.at[lid, 0])
    pltpu.sync_copy(edst_v, lists_hbm.at[lid, 1])
    idx16_v[...] = jnp.full((16,), 1, i32) * ca
    pltpu.sync_copy(idx16_v.at[pl.ds(0, 8)], lcnt_hbm.at[pl.ds(lid * 8, 8)])
    pltpu.sync_copy(cnt_v, cnt_hbm.at[wid])
    plsc.subcore_barrier()

    pltpu.sync_copy(lcnt_hbm.at[pl.ds(cid * NS * 8, 128)], c128_v)

    def _flush(n, base):
        for j in range(K // 16):
            v = pend_src[pl.ds(j * 16, 16)]
            ok = (iota16 + j * 16) < n
            idxg2[0, pl.ds(j * 16, 16)] = jnp.where(ok, v, 0)
        pltpu.async_copy(x_hbm.at[idxg2.at[0]], rows_v, sem).wait()

        def _addone(e, _):
            @pl.when(e < n)
            def _():
                rl = jnp.max(plsc.load_gather(pend_row,
                                              [jnp.full((16,), e, i32)]))
                rl = rl - base
                for cg in range(EH // 16):
                    plsc.addupdate(acc_v.at[rl, pl.ds(cg * 16, 16)],
                                   rows_v[e, pl.ds(cg * 16, 16)])
            return _
        lax.fori_loop(0, K, _addone, None)

    def _pass(p, _):
        base = sid * OPT + p * RPH

        def _zacc(i, _):
            for cg in range(EH // 16):
                acc_v[i, pl.ds(cg * 16, 16)] = zero16
            return _
        lax.fori_loop(0, RPH, _zacc, None)

        def _tbody(t, pc):
            nt = jnp.max(plsc.load_gather(c128_v,
                                          [jnp.full((16,), 8, i32) * t]))

            def _stage(c, pc):
                pltpu.sync_copy(lists_hbm.at[cid * NS + t, 0,
                                             pl.ds(c * CH, CH)],
                                esrc_v.at[pl.ds(0, CH)])
                pltpu.sync_copy(lists_hbm.at[cid * NS + t, 1,
                                             pl.ds(c * CH, CH)],
                                edst_v.at[pl.ds(0, CH)])
                rem = nt - c * CH

                def _collect(i, pc):
                    s16 = esrc_v[pl.ds(i * 16, 16)]
                    g16 = edst_v[pl.ds(i * 16, 16)]
                    mine = jnp.logical_and((i * 16 + iota16) < rem,
                                           jnp.logical_and(g16 >= base,
                                                           g16 < base + RPH))
                    plsc.store_compressed(pend_src.at[pl.ds(pc, 16)], s16,
                                          mask=mine)
                    plsc.store_compressed(pend_row.at[pl.ds(pc, 16)], g16,
                                          mask=mine)
                    pc2 = pc + jnp.max(plsc.all_reduce_population_count(mine))

                    @pl.when(pc2 >= K)
                    def _():
                        _flush(jnp.int32(K), base)
                        pend_src[pl.ds(0, 16)] = pend_src[pl.ds(K, 16)]
                        pend_row[pl.ds(0, 16)] = pend_row[pl.ds(K, 16)]
                    return jnp.where(pc2 >= K, pc2 - K, pc2)
                gc = jnp.minimum(CH // 16, (rem + 15) // 16)
                return lax.fori_loop(0, gc, _collect, pc)
            return lax.fori_loop(0, (nt + CH - 1) // CH, _stage, pc)

        pc = lax.fori_loop(0, NS, _tbody, jnp.int32(0))
        _flush(pc, base)
        pltpu.sync_copy(acc_v, a2_hbm.at[pl.ds(cid * ROWS + base, RPH)])
        return _
    lax.fori_loop(0, 2, _pass, None)


@jax.jit
def _sc_gather(espk, edst, ids, x):
    mesh = plsc.VectorSubcoreMesh(core_axis_name="c", subcore_axis_name="s")
    f = pl.kernel(
        _sc_body,
        out_type=(
            jax.ShapeDtypeStruct((NC * ROWS, EH), jnp.float32),
            jax.ShapeDtypeStruct((NW, ROWS), jnp.float32),
            jax.ShapeDtypeStruct((S, EH), jnp.float32),
            jax.ShapeDtypeStruct((S,), jnp.int32),
            jax.ShapeDtypeStruct((NW, 2, CAP), jnp.int32),
            jax.ShapeDtypeStruct((NW * 8,), jnp.int32),
        ),
        mesh=mesh,
        scratch_types=(
            pltpu.VMEM((NE,), jnp.int32),
            pltpu.VMEM((S,), jnp.int32),
            pltpu.VMEM((CAP,), jnp.int32),
            pltpu.VMEM((CAP,), jnp.int32),
            pltpu.VMEM((1, K), jnp.int32),
            pltpu.VMEM((K, EH), jnp.float32),
            pltpu.VMEM((ROWS,), jnp.float32),
            pltpu.VMEM((RPH, EH), jnp.float32),
            pltpu.VMEM((K + 16,), jnp.int32),
            pltpu.VMEM((K + 16,), jnp.int32),
            pltpu.VMEM((16,), jnp.int32),
            pltpu.VMEM((16,), jnp.int32),
            pltpu.VMEM((128,), jnp.int32),
            pltpu.SemaphoreType.DMA,
        ),
        compiler_params=pltpu.CompilerParams(needs_layout_passes=False),
    )
    return f(espk, edst, ids, x)


def _t1_body(a2_ref, cnt_ref, xg_ref, rep_ref, basis_ref, comp_ref,
             root_ref, bias_ref, e11_ref, e1b1_ref, e12_ref, e1b2_ref,
             e2w_ref, e2b_ref, p11_ref, p1b1_ref, p12_ref, p1b2_ref,
             out_ref):
    f32 = jnp.float32
    a2 = a2_ref[0:ROWS, :] + a2_ref[ROWS:2 * ROWS, :]
    xg = xg_ref[...]
    msg = jnp.zeros((S, EH), f32)
    for r in range(R):
        cnt_r = jnp.sum(cnt_ref[:, r, :], axis=0)
        inv = 1.0 / jnp.maximum(cnt_r, 1.0)
        a_r = a2[r * S:(r + 1) * S, :] * inv[:, None]
        w_r = comp_ref[r, 0] * basis_ref[0]
        for b in range(1, NBASES):
            w_r = w_r + comp_ref[r, b] * basis_ref[b]
        msg = msg + jnp.dot(a_r, w_r, preferred_element_type=f32)
    ent = (jnp.dot(xg, root_ref[...], preferred_element_type=f32)
           + bias_ref[...] + msg + xg)
    h = jnp.maximum(jnp.dot(ent, e11_ref[...], preferred_element_type=f32)
                    + e1b1_ref[...], 0.0)
    ent = ent + jnp.dot(h, e12_ref[...], preferred_element_type=f32) + e1b2_ref[...]
    ent3 = jnp.dot(ent, e2w_ref[...], preferred_element_type=f32) + e2b_ref[...]
    onehot = (rep_ref[...] == lax.broadcasted_iota(jnp.int32, (S, S), 1)
              ).astype(f32)
    pe0 = jnp.dot(onehot, ent3, preferred_element_type=f32)
    h = jnp.maximum(jnp.dot(pe0, p11_ref[...], preferred_element_type=f32)
                    + p1b1_ref[...], 0.0)
    out_ref[...] = (pe0 + jnp.dot(h, p12_ref[...], preferred_element_type=f32)
                    + p1b2_ref[...])


def _t2_body(pe1_ref, w_ref, b_ref, out_ref):
    y = jnp.dot(pe1_ref[...], w_ref[...], preferred_element_type=jnp.float32)
    y = y + b_ref[0]
    for h in range(NH):
        out_ref[0, 0, :, h, :, :] = y[:, h * HD:(h + 1) * HD].reshape(B, LENT, HD)


def kernel(node_embeds, basis, comp, root, bias,
           ep1_w1, ep1_b1, ep1_w2, ep1_b2, ep2_w, ep2_b,
           pp1_w1, pp1_b1, pp1_w2, pp1_b2, pp2_w, pp2_b,
           edge_index, edge_type, entity_ids, rec_labels):
    ids = entity_ids.reshape(-1)
    espk = edge_index[0] * 16 + edge_type
    a2, cnt, xg, rep, _, _ = _sc_gather(espk, edge_index[1], ids, node_embeds)

    vspec = pl.BlockSpec(memory_space=pltpu.VMEM)
    pe1 = pl.pallas_call(
        _t1_body,
        in_specs=[vspec, vspec, vspec, vspec, vspec,
                  pl.BlockSpec(memory_space=pltpu.SMEM),
                  vspec, vspec, vspec, vspec, vspec, vspec, vspec, vspec,
                  vspec, vspec, vspec, vspec],
        out_specs=vspec,
        out_shape=jax.ShapeDtypeStruct((S, HID), jnp.float32),
    )(a2, cnt.reshape(NW, R, S), xg, rep.reshape(S, 1), basis, comp,
      root, bias.reshape(1, EH),
      ep1_w1, ep1_b1.reshape(1, -1), ep1_w2, ep1_b2.reshape(1, -1),
      ep2_w, ep2_b.reshape(1, -1),
      pp1_w1, pp1_b1.reshape(1, -1), pp1_w2, pp1_b2.reshape(1, -1))

    out = pl.pallas_call(
        _t2_body,
        grid=(NL * NB,),
        in_specs=[
            pl.BlockSpec((S, HID), lambda i: (0, 0)),
            pl.BlockSpec((HID, HID), lambda i: (0, i)),
            pl.BlockSpec((1, 1, HID), lambda i: (i, 0, 0)),
        ],
        out_specs=pl.BlockSpec((1, 1, B, NH, LENT, HD),
                               lambda i: (i // 2, i % 2, 0, 0, 0, 0)),
        out_shape=jax.ShapeDtypeStruct((NL, NB, B, NH, LENT, HD), jnp.float32),
    )(pe1, pp2_w, pp2_b.reshape(NL * NB, 1, HID))
    return out


# speedup vs baseline: 78.2356x; 1.0004x over previous
import jax
import jax.numpy as jnp
from jax import lax
from jax.experimental import pallas as pl
from jax.experimental.pallas import tpu as pltpu, tpu_sc as plsc

HID = 768
EH = 384
NL = 12
NB = 2
NH = 12
HD = 64
NE = 10000
E = 320000
R = 9
NBASES = 8
B = 16
LENT = 32
S = B * LENT

NC = 2
NS = 16
NW = NC * NS
EPT = E // NW
K = 64
ROWS = R * S
OPT = ROWS // NS
RPH = OPT // 2
CH = 1024
CAP = EPT + CH - EPT % CH


def _sc_body(espk_hbm, edst_hbm, ids_hbm, x_hbm,
             a2_hbm, cnt_hbm, xg_hbm, rep_hbm, lists_hbm, lcnt_hbm,
             slot_v, ids_v, esrc_v, edst_v,
             idxg2, rows_v, cnt_v, acc_v, pend_src, pend_row,
             idx16_v, rep_v, c128_v, sem):
    cid = lax.axis_index("c")
    sid = lax.axis_index("s")
    wid = sid * NC + cid
    lid = cid * NS + sid
    i32 = jnp.int32
    f32 = jnp.float32
    zero16 = jnp.zeros((16,), f32)
    iota16 = lax.broadcasted_iota(i32, (16,), 0)

    pltpu.sync_copy(ids_hbm, ids_v)

    def _init_slot(i, _):
        slot_v[pl.ds(i * 16, 16)] = jnp.full((16,), -1, i32)
        return _
    lax.fori_loop(0, NE // 16, _init_slot, None)

    def _scatter_ids(i, _):
        idv = ids_v[pl.ds(i * 16, 16)]
        plsc.store_scatter(slot_v, [idv], iota16 + i * 16)
        return _
    lax.fori_loop(0, S // 16, _scatter_ids, None)

    def _zero_cnt(i, _):
        cnt_v[pl.ds(i * 16, 16)] = zero16
        return _
    lax.fori_loop(0, ROWS // 16, _zero_cnt, None)

    myids = ids_v[pl.ds(wid * 16, 16)]
    idx16_v[...] = myids
    pltpu.async_copy(x_hbm.at[idx16_v], rows_v.at[pl.ds(0, 16)], sem).wait()
    pltpu.sync_copy(rows_v.at[pl.ds(0, 16)], xg_hbm.at[pl.ds(wid * 16, 16)])
    rep_v[...] = plsc.load_gather(slot_v, [myids])
    pltpu.sync_copy(rep_v, rep_hbm.at[pl.ds(wid * 16, 16)])

    pltpu.sync_copy(espk_hbm.at[pl.ds(wid * EPT, EPT)], esrc_v.at[pl.ds(0, EPT)])
    pltpu.sync_copy(edst_hbm.at[pl.ds(wid * EPT, EPT)], edst_v.at[pl.ds(0, EPT)])

    ones16 = jnp.full((16,), 1.0, f32)

    def _scan(i, ca):
        st16 = esrc_v[pl.ds(i * 16, 16)]
        s16 = lax.shift_right_logical(st16, 4)
        t16 = jnp.bitwise_and(st16, 15)
        d16 = edst_v[pl.ds(i * 16, 16)]
        sl16 = plsc.load_gather(slot_v, [d16])
        m = sl16 >= 0
        slotidx = t16 * S + sl16
        plsc.addupdate_scatter(cnt_v, [jnp.where(m, slotidx, 0)], ones16,
                               mask=m)
        plsc.store_compressed(esrc_v.at[pl.ds(ca, 16)], s16, mask=m)
        plsc.store_compressed(edst_v.at[pl.ds(ca, 16)], slotidx, mask=m)
        return ca + jnp.max(plsc.all_reduce_population_count(m))
    ca = lax.fori_loop(0, EPT // 16, _scan, jnp.int32(0))

    pltpu.sync_copy(esrc_v, lists_hbm.at[lid, 0])
    pltpu.sync_copy(edst_v, lists_hbm.at[lid, 1])
    idx16_v[...] = jnp.full((16,), 1, i32) * ca
    pltpu.sync_copy(idx16_v.at[pl.ds(0, 8)], lcnt_hbm.at[pl.ds(lid * 8, 8)])
    pltpu.sync_copy(cnt_v, cnt_hbm.at[wid])
    plsc.subcore_barrier()

    pltpu.sync_copy(lcnt_hbm.at[pl.ds(cid * NS * 8, 128)], c128_v)

    def _flush(n, base):
        for j in range(K // 16):
            v = pend_src[pl.ds(j * 16, 16)]
            ok = (iota16 + j * 16) < n
            idxg2[0, pl.ds(j * 16, 16)] = jnp.where(ok, v, 0)
        pltpu.async_copy(x_hbm.at[idxg2.at[0]], rows_v, sem).wait()

        def _addone(e, _):
            @pl.when(e < n)
            def _():
                rl = jnp.max(plsc.load_gather(pend_row,
                                              [jnp.full((16,), e, i32)]))
                rl = rl - base
                for cg in range(EH // 16):
                    plsc.addupdate(acc_v.at[rl, pl.ds(cg * 16, 16)],
                                   rows_v[e, pl.ds(cg * 16, 16)])
            return _
        lax.fori_loop(0, K, _addone, None)

    def _pass(p, _):
        base = sid * OPT + p * RPH

        def _zacc(i, _):
            for cg in range(EH // 16):
                acc_v[i, pl.ds(cg * 16, 16)] = zero16
            return _
        lax.fori_loop(0, RPH, _zacc, None)

        def _tbody(t, pc):
            nt = jnp.max(plsc.load_gather(c128_v,
                                          [jnp.full((16,), 8, i32) * t]))

            def _stage(c, pc):
                pltpu.sync_copy(lists_hbm.at[cid * NS + t, 0,
                                             pl.ds(c * CH, CH)],
                                esrc_v.at[pl.ds(0, CH)])
                pltpu.sync_copy(lists_hbm.at[cid * NS + t, 1,
                                             pl.ds(c * CH, CH)],
                                edst_v.at[pl.ds(0, CH)])
                rem = nt - c * CH

                def _collect(i, pc):
                    s16 = esrc_v[pl.ds(i * 16, 16)]
                    g16 = edst_v[pl.ds(i * 16, 16)]
                    mine = jnp.logical_and((i * 16 + iota16) < rem,
                                           jnp.logical_and(g16 >= base,
                                                           g16 < base + RPH))
                    plsc.store_compressed(pend_src.at[pl.ds(pc, 16)], s16,
                                          mask=mine)
                    plsc.store_compressed(pend_row.at[pl.ds(pc, 16)], g16,
                                          mask=mine)
                    pc2 = pc + jnp.max(plsc.all_reduce_population_count(mine))

                    @pl.when(pc2 >= K)
                    def _():
                        _flush(jnp.int32(K), base)
                        pend_src[pl.ds(0, 16)] = pend_src[pl.ds(K, 16)]
                        pend_row[pl.ds(0, 16)] = pend_row[pl.ds(K, 16)]
                    return jnp.where(pc2 >= K, pc2 - K, pc2)
                gc = jnp.minimum(CH // 16, (rem + 15) // 16)
                return lax.fori_loop(0, gc, _collect, pc)
            return lax.fori_loop(0, (nt + CH - 1) // CH, _stage, pc)

        pc = lax.fori_loop(0, NS, _tbody, jnp.int32(0))
        _flush(pc, base)
        pltpu.sync_copy(acc_v, a2_hbm.at[pl.ds(cid * ROWS + base, RPH)])
        return _
    lax.fori_loop(0, 2, _pass, None)


@jax.jit
def _sc_gather(espk, edst, ids, x):
    mesh = plsc.VectorSubcoreMesh(core_axis_name="c", subcore_axis_name="s")
    f = pl.kernel(
        _sc_body,
        out_type=(
            jax.ShapeDtypeStruct((NC * ROWS, EH), jnp.float32),
            jax.ShapeDtypeStruct((NW, ROWS), jnp.float32),
            jax.ShapeDtypeStruct((S, EH), jnp.float32),
            jax.ShapeDtypeStruct((S,), jnp.int32),
            jax.ShapeDtypeStruct((NW, 2, CAP), jnp.int32),
            jax.ShapeDtypeStruct((NW * 8,), jnp.int32),
        ),
        mesh=mesh,
        scratch_types=(
            pltpu.VMEM((NE,), jnp.int32),
            pltpu.VMEM((S,), jnp.int32),
            pltpu.VMEM((CAP,), jnp.int32),
            pltpu.VMEM((CAP,), jnp.int32),
            pltpu.VMEM((1, K), jnp.int32),
            pltpu.VMEM((K, EH), jnp.float32),
            pltpu.VMEM((ROWS,), jnp.float32),
            pltpu.VMEM((RPH, EH), jnp.float32),
            pltpu.VMEM((K + 16,), jnp.int32),
            pltpu.VMEM((K + 16,), jnp.int32),
            pltpu.VMEM((16,), jnp.int32),
            pltpu.VMEM((16,), jnp.int32),
            pltpu.VMEM((128,), jnp.int32),
            pltpu.SemaphoreType.DMA,
        ),
        compiler_params=pltpu.CompilerParams(needs_layout_passes=False),
    )
    return f(espk, edst, ids, x)


def _t1_body(a2_ref, cnt_ref, xg_ref, rep_ref, basis_ref, comp_ref,
             root_ref, bias_ref, e11_ref, e1b1_ref, e12_ref, e1b2_ref,
             e2w_ref, e2b_ref, p11_ref, p1b1_ref, p12_ref, p1b2_ref,
             out_ref):
    f32 = jnp.float32
    a2 = a2_ref[0:ROWS, :] + a2_ref[ROWS:2 * ROWS, :]
    xg = xg_ref[...]
    msg = jnp.zeros((S, EH), f32)
    for r in range(R):
        cnt_r = jnp.sum(cnt_ref[:, r, :], axis=0)
        inv = 1.0 / jnp.maximum(cnt_r, 1.0)
        a_r = a2[r * S:(r + 1) * S, :] * inv[:, None]
        w_r = comp_ref[r, 0] * basis_ref[0]
        for b in range(1, NBASES):
            w_r = w_r + comp_ref[r, b] * basis_ref[b]
        msg = msg + jnp.dot(a_r, w_r, preferred_element_type=f32)
    ent = (jnp.dot(xg, root_ref[...], preferred_element_type=f32)
           + bias_ref[...] + msg + xg)
    h = jnp.maximum(jnp.dot(ent, e11_ref[...], preferred_element_type=f32)
                    + e1b1_ref[...], 0.0)
    ent = ent + jnp.dot(h, e12_ref[...], preferred_element_type=f32) + e1b2_ref[...]
    ent3 = jnp.dot(ent, e2w_ref[...], preferred_element_type=f32) + e2b_ref[...]
    onehot = (rep_ref[...] == lax.broadcasted_iota(jnp.int32, (S, S), 1)
              ).astype(f32)
    pe0 = jnp.dot(onehot, ent3, preferred_element_type=f32)
    h = jnp.maximum(jnp.dot(pe0, p11_ref[...], preferred_element_type=f32)
                    + p1b1_ref[...], 0.0)
    out_ref[...] = (pe0 + jnp.dot(h, p12_ref[...], preferred_element_type=f32)
                    + p1b2_ref[...])


def _t2_body(pe1_ref, w_ref, b_ref, out_ref):
    y = jnp.dot(pe1_ref[...], w_ref[...], preferred_element_type=jnp.float32)
    y = y + b_ref[0]
    for h in range(NH):
        out_ref[0, 0, :, h, :, :] = y[:, h * HD:(h + 1) * HD].reshape(B, LENT, HD)


def kernel(node_embeds, basis, comp, root, bias,
           ep1_w1, ep1_b1, ep1_w2, ep1_b2, ep2_w, ep2_b,
           pp1_w1, pp1_b1, pp1_w2, pp1_b2, pp2_w, pp2_b,
           edge_index, edge_type, entity_ids, rec_labels):
    ids = entity_ids.reshape(-1)
    espk = edge_index[0] * 16 + edge_type
    a2, cnt, xg, rep, _, _ = _sc_gather(espk, edge_index[1], ids, node_embeds)

    vspec = pl.BlockSpec(memory_space=pltpu.VMEM)
    pe1 = pl.pallas_call(
        _t1_body,
        in_specs=[vspec, vspec, vspec, vspec, vspec,
                  pl.BlockSpec(memory_space=pltpu.SMEM),
                  vspec, vspec, vspec, vspec, vspec, vspec, vspec, vspec,
                  vspec, vspec, vspec, vspec],
        out_specs=vspec,
        out_shape=jax.ShapeDtypeStruct((S, HID), jnp.float32),
    )(a2, cnt.reshape(NW, R, S), xg, rep.reshape(S, 1), basis, comp,
      root, bias.reshape(1, EH),
      ep1_w1, ep1_b1.reshape(1, -1), ep1_w2, ep1_b2.reshape(1, -1),
      ep2_w, ep2_b.reshape(1, -1),
      pp1_w1, pp1_b1.reshape(1, -1), pp1_w2, pp1_b2.reshape(1, -1))

    out = pl.pallas_call(
        _t2_body,
        grid=(NL * NB,),
        in_specs=[
            pl.BlockSpec((S, HID), lambda i: (0, 0)),
            pl.BlockSpec((HID, HID), lambda i: (0, i)),
            pl.BlockSpec((1, 1, HID), lambda i: (i, 0, 0)),
        ],
        out_specs=pl.BlockSpec((1, 1, B, NH, LENT, HD),
                               lambda i: (i // 2, i % 2, 0, 0, 0, 0)),
        out_shape=jax.ShapeDtypeStruct((NL, NB, B, NH, LENT, HD), jnp.float32),
    )(pe1, pp2_w, pp2_b.reshape(NL * NB, 1, HID))
    return out
